# baseline (speedup 1.0000x reference)
import functools

import jax
import jax.numpy as jnp
from jax import lax
from jax.experimental import pallas as pl
from jax.experimental.pallas import tpu as pltpu
from jax.experimental.pallas import tpu_sc as plsc

VOCAB = 1000
VPAD = 1024
NC, NS, LANES = 2, 16, 16
NW = NC * NS
B, T = 1024, 50
NBT = B // 128
NVT = VOCAB // 8
UNITS_W = (T * NBT * 2) // NW
POS_W = (B * T) // NW
GCH = 80


def _lse_body(tab_ref, out_ref):
    x = tab_ref[...]
    m = jnp.max(x, axis=1)
    s = jnp.sum(jnp.exp(x - m[:, None]), axis=1)
    out_ref[...] = m + jnp.log(s)


_sc_mesh = plsc.VectorSubcoreMesh(core_axis_name="c", subcore_axis_name="s",
                                  num_cores=NC, num_subcores=NS)


@functools.partial(
    pl.kernel,
    out_type=jax.ShapeDtypeStruct((T, NVT, 8, B), jnp.float32),
    mesh=_sc_mesh,
    compiler_params=pltpu.CompilerParams(use_tc_tiling_on_sc=True,
                                         needs_layout_passes=False),
    scratch_types=[
        pltpu.VMEM((128,), jnp.int32),
        pltpu.VMEM((16, 8, 128), jnp.float32),
        pltpu.VMEM((16, 8, 128), jnp.float32),
        pltpu.VMEM((64, 8, 128), jnp.float32),
        pltpu.SemaphoreType.DMA,
        pltpu.SemaphoreType.DMA,
        pltpu.SemaphoreType.DMA,
    ],
)
def _sc_gather_t(tab3_hbm, idx_hbm, out_hbm,
                 idx_v, rg0, rg1, stg, semg0, semg1, semo):
    wid = lax.axis_index("s") * NC + lax.axis_index("c")
    rgs = (rg0, rg1)
    sems = (semg0, semg1)
    biota = lax.iota(jnp.int32, LANES)

    def unit_coords(u):
        t = u // (NBT * 2)
        rem = u % (NBT * 2)
        return t, rem // 2, rem % 2

    def out_slab(t, bt, vh, nvt):
        return out_hbm.at[t, pl.ds(vh * 64, nvt), :, pl.ds(bt * 128, 128)]

    def gather_desc(g, p):
        return pltpu.make_async_copy(
            tab3_hbm.at[idx_v.at[pl.ds(g * LANES, LANES)]], rgs[p], sems[p])

    @pl.loop(0, UNITS_W)
    def _unit(k):
        u = wid + NW * k
        t, bt, vh = unit_coords(u)
        nvt = jnp.where(vh == 0, 64, NVT - 64)
        pltpu.sync_copy(idx_hbm.at[pl.ds(t * B + bt * 128, 128)], idx_v)

        @pl.loop(0, 8)
        def _group(g):
            if True:
                rg = rgs[0]
                pltpu.sync_copy(
                    tab3_hbm.at[idx_v.at[pl.ds(g * LANES, LANES)]], rg)

                @functools.partial(plsc.parallel_loop, 0, 64, unroll=2)
                def _vtb(vtl):
                    sv = jnp.full((LANES,), vh * 4 + vtl // 16, jnp.int32)
                    lb = (vtl % 16) * 8
                    for vi in range(8):
                        lv = jnp.full((LANES,), lb + vi, jnp.int32)
                        col = plsc.load_gather(rg, [biota, sv, lv])
                        stg[vtl, vi, pl.ds(g * LANES, LANES)] = col

        pltpu.sync_copy(stg.at[pl.ds(0, nvt)], out_slab(t, bt, vh, nvt))


@functools.partial(
    pl.kernel,
    out_type=jax.ShapeDtypeStruct((NW, LANES), jnp.float32),
    mesh=_sc_mesh,
    compiler_params=pltpu.CompilerParams(use_tc_tiling_on_sc=False,
                                         needs_layout_passes=False),
    scratch_types=[
        pltpu.VMEM((POS_W,), jnp.int32),
        pltpu.VMEM((POS_W,), jnp.int32),
        pltpu.VMEM((POS_W,), jnp.int32),
        pltpu.VMEM((POS_W,), jnp.float32),
        pltpu.VMEM((VOCAB,), jnp.float32),
        pltpu.VMEM((LANES,), jnp.float32),
        pltpu.SemaphoreType.DMA,
    ],
)
def _sc_loss(tabflat_hbm, idx_hbm, tgt_hbm, lse_hbm, part_hbm,
             idx_v, tgt_v, fi_v, tv_v, lse_v, acc_v, sem):
    wid = lax.axis_index("s") * NC + lax.axis_index("c")
    base = wid * POS_W
    pltpu.sync_copy(idx_hbm.at[pl.ds(base, POS_W)], idx_v)
    pltpu.sync_copy(tgt_hbm.at[pl.ds(base, POS_W)], tgt_v)
    pltpu.sync_copy(lse_hbm, lse_v)

    @pl.loop(0, POS_W, step=LANES)
    def _flat(o):
        fi_v[pl.ds(o, LANES)] = (idx_v[pl.ds(o, LANES)] * VOCAB
                                 + tgt_v[pl.ds(o, LANES)])

    descs = [
        pltpu.make_async_copy(
            tabflat_hbm.at[fi_v.at[pl.ds(c * GCH, GCH)]],
            tv_v.at[pl.ds(c * GCH, GCH)], sem)
        for c in range(POS_W // GCH)
    ]
    for d in descs:
        d.start()
    for d in descs:
        d.wait()

    def acc_step(o, acc):
        ls = plsc.load_gather(lse_v, [idx_v[pl.ds(o, LANES)]])
        return acc + (ls - tv_v[pl.ds(o, LANES)])

    acc = pl.loop(0, POS_W, step=LANES,
                  init_carry=jnp.zeros((LANES,), jnp.float32))(acc_step)
    acc_v[...] = acc
    pltpu.sync_copy(acc_v, part_hbm.at[wid])


@jax.jit
def kernel(idx, targets, table):
    table = table.astype(jnp.float32)
    tab3 = jnp.pad(table, ((0, 0), (0, VPAD - VOCAB))).reshape(VOCAB, 8, 128)
    idx_bt = idx.astype(jnp.int32).T.reshape(-1)
    idx_f = idx.astype(jnp.int32).reshape(-1)
    tgt_f = targets.astype(jnp.int32).reshape(-1)
    lse = pl.pallas_call(
        _lse_body,
        out_shape=jax.ShapeDtypeStruct((VOCAB,), jnp.float32),
    )(table)
    out4d = _sc_gather_t(tab3, idx_bt)
    logits = out4d.reshape(T, VOCAB, B).transpose(2, 0, 1)
    parts = _sc_loss(table.reshape(-1), idx_f, tgt_f, lse)
    loss = jnp.sum(parts) / jnp.float32(B * T)
    return logits, loss


# speedup vs baseline: 1.2659x; 1.2659x over previous
import functools

import jax
import jax.numpy as jnp
from jax import lax
from jax.experimental import pallas as pl
from jax.experimental.pallas import tpu as pltpu
from jax.experimental.pallas import tpu_sc as plsc

VOCAB = 1000
VPAD = 1024
NC, NS, LANES = 2, 16, 16
NW = NC * NS
B, T = 1024, 50
NBT = B // 128
NVT = VOCAB // 8
UNITS_W = (T * NBT * 2) // NW
POS_W = (B * T) // NW
GCH = 80


def _lse_body(tab_ref, out_ref):
    x = tab_ref[...]
    m = jnp.max(x, axis=1)
    s = jnp.sum(jnp.exp(x - m[:, None]), axis=1)
    out_ref[...] = m + jnp.log(s)


_sc_mesh = plsc.VectorSubcoreMesh(core_axis_name="c", subcore_axis_name="s",
                                  num_cores=NC, num_subcores=NS)


@functools.partial(
    pl.kernel,
    out_type=jax.ShapeDtypeStruct((T, NVT, 8, B), jnp.float32),
    mesh=_sc_mesh,
    compiler_params=pltpu.CompilerParams(use_tc_tiling_on_sc=True,
                                         needs_layout_passes=False),
    scratch_types=[
        pltpu.VMEM((128,), jnp.int32),
        pltpu.VMEM((16, 8, 128), jnp.float32),
        pltpu.VMEM((16, 8, 128), jnp.float32),
        pltpu.VMEM((64, 8, 128), jnp.float32),
        pltpu.SemaphoreType.DMA,
        pltpu.SemaphoreType.DMA,
        pltpu.SemaphoreType.DMA,
    ],
)
def _sc_gather_t(tab3_hbm, idx_hbm, out_hbm,
                 idx_v, rg0, rg1, stg, semg0, semg1, semo):
    wid = lax.axis_index("s") * NC + lax.axis_index("c")
    rgs = (rg0, rg1)
    sems = (semg0, semg1)
    biota = lax.iota(jnp.int32, LANES)

    def unit_coords(u):
        t = u // (NBT * 2)
        rem = u % (NBT * 2)
        return t, rem // 2, rem % 2

    def out_slab(t, bt, vh, nvt):
        return out_hbm.at[t, pl.ds(vh * 64, nvt), :, pl.ds(bt * 128, 128)]

    def gather_desc(g, p):
        return pltpu.make_async_copy(
            tab3_hbm.at[idx_v.at[pl.ds(g * LANES, LANES)]], rgs[p], sems[p])

    @pl.loop(0, UNITS_W)
    def _unit(k):
        u = wid + NW * k
        t, bt, vh = unit_coords(u)
        nvt = jnp.where(vh == 0, 64, NVT - 64)
        pltpu.sync_copy(idx_hbm.at[pl.ds(t * B + bt * 128, 128)], idx_v)

        for gp in range(4):
            d0 = gather_desc(2 * gp, 0)
            d1 = gather_desc(2 * gp + 1, 1)
            d0.start()
            d1.start()
            d0.wait()
            d1.wait()
            for p in range(2):
                g = 2 * gp + p
                rg = rgs[p]

                @functools.partial(plsc.parallel_loop, 0, 64, unroll=2)
                def _vtb(vtl):
                    sv = jnp.full((LANES,), vh * 4 + vtl // 16, jnp.int32)
                    lb = (vtl % 16) * 8
                    for vi in range(8):
                        lv = jnp.full((LANES,), lb + vi, jnp.int32)
                        col = plsc.load_gather(rg, [biota, sv, lv])
                        stg[vtl, vi, pl.ds(g * LANES, LANES)] = col

        pltpu.sync_copy(stg.at[pl.ds(0, nvt)], out_slab(t, bt, vh, nvt))


@functools.partial(
    pl.kernel,
    out_type=jax.ShapeDtypeStruct((NW, LANES), jnp.float32),
    mesh=_sc_mesh,
    compiler_params=pltpu.CompilerParams(use_tc_tiling_on_sc=False,
                                         needs_layout_passes=False),
    scratch_types=[
        pltpu.VMEM((POS_W,), jnp.int32),
        pltpu.VMEM((POS_W,), jnp.int32),
        pltpu.VMEM((POS_W,), jnp.int32),
        pltpu.VMEM((POS_W,), jnp.float32),
        pltpu.VMEM((VOCAB,), jnp.float32),
        pltpu.VMEM((LANES,), jnp.float32),
        pltpu.SemaphoreType.DMA,
    ],
)
def _sc_loss(tabflat_hbm, idx_hbm, tgt_hbm, lse_hbm, part_hbm,
             idx_v, tgt_v, fi_v, tv_v, lse_v, acc_v, sem):
    wid = lax.axis_index("s") * NC + lax.axis_index("c")
    base = wid * POS_W
    pltpu.sync_copy(idx_hbm.at[pl.ds(base, POS_W)], idx_v)
    pltpu.sync_copy(tgt_hbm.at[pl.ds(base, POS_W)], tgt_v)
    pltpu.sync_copy(lse_hbm, lse_v)

    @pl.loop(0, POS_W, step=LANES)
    def _flat(o):
        fi_v[pl.ds(o, LANES)] = (idx_v[pl.ds(o, LANES)] * VOCAB
                                 + tgt_v[pl.ds(o, LANES)])

    descs = [
        pltpu.make_async_copy(
            tabflat_hbm.at[fi_v.at[pl.ds(c * GCH, GCH)]],
            tv_v.at[pl.ds(c * GCH, GCH)], sem)
        for c in range(POS_W // GCH)
    ]
    for d in descs:
        d.start()
    for d in descs:
        d.wait()

    def acc_step(o, acc):
        ls = plsc.load_gather(lse_v, [idx_v[pl.ds(o, LANES)]])
        return acc + (ls - tv_v[pl.ds(o, LANES)])

    acc = pl.loop(0, POS_W, step=LANES,
                  init_carry=jnp.zeros((LANES,), jnp.float32))(acc_step)
    acc_v[...] = acc
    pltpu.sync_copy(acc_v, part_hbm.at[wid])


@jax.jit
def kernel(idx, targets, table):
    table = table.astype(jnp.float32)
    tab3 = jnp.pad(table, ((0, 0), (0, VPAD - VOCAB))).reshape(VOCAB, 8, 128)
    idx_bt = idx.astype(jnp.int32).T.reshape(-1)
    idx_f = idx.astype(jnp.int32).reshape(-1)
    tgt_f = targets.astype(jnp.int32).reshape(-1)
    lse = pl.pallas_call(
        _lse_body,
        out_shape=jax.ShapeDtypeStruct((VOCAB,), jnp.float32),
    )(table)
    out4d = _sc_gather_t(tab3, idx_bt)
    logits = out4d.reshape(T, VOCAB, B).transpose(2, 0, 1)
    parts = _sc_loss(table.reshape(-1), idx_f, tgt_f, lse)
    loss = jnp.sum(parts) / jnp.float32(B * T)
    return logits, loss
